# baseline (speedup 1.0000x reference)
import functools

import jax
import jax.numpy as jnp
from jax.experimental import pallas as pl
from jax.experimental.pallas import tpu as pltpu


def _linear_kernel(xin_ref, w_ref, bias_ref, xt_ref, *, B, F):
    wt = w_ref[...].T
    for bb in range(B):
        sl = slice(bb * F, (bb + 1) * F)
        y = jnp.dot(xin_ref[:, sl], wt, preferred_element_type=jnp.float32)
        xt_ref[:, sl] = (y + bias_ref[...]).astype(jnp.bfloat16)


def _matmul_kernel(xt_ref, p_ref, mask_ref, pos_ref, out_ref, *, B, F, n):
    p = p_ref[...].astype(jnp.bfloat16)
    blkN = p.shape[0]
    col = jax.lax.broadcasted_iota(jnp.int32, (blkN, n), 1)
    onehot = (col == pos_ref[...]).astype(jnp.bfloat16)
    p_eff = jnp.where(mask_ref[...] > 0.5, onehot, p)
    y = jnp.dot(p_eff, xt_ref[...], preferred_element_type=jnp.float32)
    out_ref[...] = y.reshape(blkN, B, F).transpose(1, 0, 2)


def kernel(input, P, sample, W, b):
    Bz, n, F = input.shape
    N = P.shape[0]
    BF = Bz * F
    blkN = 512

    xin = input.transpose(1, 0, 2).reshape(n, BF)
    mask = jnp.zeros((N, 1), jnp.float32).at[sample, 0].set(1.0)
    pos = jnp.zeros((N, 1), jnp.int32).at[sample, 0].set(
        jnp.arange(n, dtype=jnp.int32))

    xt = pl.pallas_call(
        functools.partial(_linear_kernel, B=Bz, F=F),
        in_specs=[
            pl.BlockSpec((n, BF), lambda: (0, 0)),
            pl.BlockSpec((F, F), lambda: (0, 0)),
            pl.BlockSpec((1, F), lambda: (0, 0)),
        ],
        out_specs=pl.BlockSpec((n, BF), lambda: (0, 0)),
        out_shape=jax.ShapeDtypeStruct((n, BF), jnp.bfloat16),
    )(xin, W, b.reshape(1, F))

    out = pl.pallas_call(
        functools.partial(_matmul_kernel, B=Bz, F=F, n=n),
        grid=(N // blkN,),
        in_specs=[
            pl.BlockSpec((n, BF), lambda i: (0, 0)),
            pl.BlockSpec((blkN, n), lambda i: (i, 0)),
            pl.BlockSpec((blkN, 1), lambda i: (i, 0)),
            pl.BlockSpec((blkN, 1), lambda i: (i, 0)),
        ],
        out_specs=pl.BlockSpec((Bz, blkN, F), lambda i: (0, i, 0)),
        out_shape=jax.ShapeDtypeStruct((Bz, N, F), jnp.float32),
    )(xt, P, mask, pos)

    return out


# speedup vs baseline: 1.1589x; 1.1589x over previous
import functools

import jax
import jax.numpy as jnp
from jax.experimental import pallas as pl
from jax.experimental.pallas import tpu as pltpu


def _linear_kernel(xin_ref, w_ref, bias_ref, xt_ref, *, B, F):
    wt = w_ref[...].T
    for bb in range(B):
        sl = slice(bb * F, (bb + 1) * F)
        y = jnp.dot(xin_ref[:, sl], wt, preferred_element_type=jnp.float32)
        xt_ref[:, sl] = (y + bias_ref[...]).astype(jnp.bfloat16)


def _matmul_kernel(xt_ref, p_ref, mask_ref, pos_ref, out_ref, *, n):
    p = p_ref[...].astype(jnp.bfloat16)
    blkN = p.shape[0]
    col = jax.lax.broadcasted_iota(jnp.int32, (blkN, n), 1)
    onehot = (col == pos_ref[...]).astype(jnp.bfloat16)
    p_eff = jnp.where(mask_ref[...] > 0.5, onehot, p)
    out_ref[...] = jnp.dot(p_eff, xt_ref[...], preferred_element_type=jnp.float32)


def kernel(input, P, sample, W, b):
    Bz, n, F = input.shape
    N = P.shape[0]
    BF = Bz * F
    blkN = 1024

    xin = input.transpose(1, 0, 2).reshape(n, BF)
    mask = jnp.zeros((N, 1), jnp.float32).at[sample, 0].set(1.0)
    pos = jnp.zeros((N, 1), jnp.int32).at[sample, 0].set(
        jnp.arange(n, dtype=jnp.int32))

    xt = pl.pallas_call(
        functools.partial(_linear_kernel, B=Bz, F=F),
        in_specs=[
            pl.BlockSpec((n, BF), lambda: (0, 0)),
            pl.BlockSpec((F, F), lambda: (0, 0)),
            pl.BlockSpec((1, F), lambda: (0, 0)),
        ],
        out_specs=pl.BlockSpec((n, BF), lambda: (0, 0)),
        out_shape=jax.ShapeDtypeStruct((n, BF), jnp.bfloat16),
    )(xin, W, b.reshape(1, F))

    y2 = pl.pallas_call(
        functools.partial(_matmul_kernel, n=n),
        grid=(N // blkN,),
        in_specs=[
            pl.BlockSpec((n, BF), lambda i: (0, 0)),
            pl.BlockSpec((blkN, n), lambda i: (i, 0)),
            pl.BlockSpec((blkN, 1), lambda i: (i, 0)),
            pl.BlockSpec((blkN, 1), lambda i: (i, 0)),
        ],
        out_specs=pl.BlockSpec((blkN, BF), lambda i: (i, 0)),
        out_shape=jax.ShapeDtypeStruct((N, BF), jnp.float32),
    )(xt, P, mask, pos)

    return y2.reshape(N, Bz, F).transpose(1, 0, 2)


# speedup vs baseline: 1.2857x; 1.1094x over previous
import functools

import jax
import jax.numpy as jnp
from jax.experimental import pallas as pl
from jax.experimental.pallas import tpu as pltpu


def _linear_kernel(xin_ref, w_ref, bias_ref, xt_ref, *, n, F):
    wt = w_ref[...].T
    y = jnp.dot(xin_ref[...], wt, preferred_element_type=jnp.float32)
    y = (y + bias_ref[...]).astype(jnp.bfloat16)
    xt_ref[:, :F] = y[:n]
    xt_ref[:, F:] = y[n:]


def _matmul_kernel(xt_ref, smp_ref, p_ref, out_ref, *, n, blkN):
    i = pl.program_id(0)
    p = p_ref[...].astype(jnp.bfloat16)
    rows = i * blkN + jax.lax.broadcasted_iota(jnp.int32, (blkN, n), 0)
    s_onehot = (smp_ref[...] == rows)
    rowmask = jnp.any(s_onehot, axis=1, keepdims=True)
    p_eff = jnp.where(rowmask, s_onehot.astype(jnp.bfloat16), p)
    out_ref[...] = jnp.dot(p_eff, xt_ref[...],
                           preferred_element_type=jnp.float32)


def kernel(input, P, sample, W, b):
    Bz, n, F = input.shape
    N = P.shape[0]
    BF = Bz * F
    blkN = 512

    xin = input.reshape(Bz * n, F)

    xt = pl.pallas_call(
        functools.partial(_linear_kernel, n=n, F=F),
        grid=(Bz // 2,),
        in_specs=[
            pl.BlockSpec((2 * n, F), lambda bb: (bb, 0)),
            pl.BlockSpec((F, F), lambda bb: (0, 0)),
            pl.BlockSpec((1, F), lambda bb: (0, 0)),
        ],
        out_specs=pl.BlockSpec((n, 2 * F), lambda bb: (0, bb)),
        out_shape=jax.ShapeDtypeStruct((n, BF), jnp.bfloat16),
    )(xin, W, b.reshape(1, F))

    y2 = pl.pallas_call(
        functools.partial(_matmul_kernel, n=n, blkN=blkN),
        grid=(N // blkN,),
        in_specs=[
            pl.BlockSpec((n, BF), lambda i: (0, 0)),
            pl.BlockSpec((1, n), lambda i: (0, 0)),
            pl.BlockSpec((blkN, n), lambda i: (i, 0)),
        ],
        out_specs=pl.BlockSpec((blkN, BF), lambda i: (i, 0)),
        out_shape=jax.ShapeDtypeStruct((N, BF), jnp.float32),
    )(xt, sample.reshape(1, n).astype(jnp.int32), P)

    return y2.reshape(N, Bz, F).transpose(1, 0, 2)


# speedup vs baseline: 1.3985x; 1.0877x over previous
import functools

import jax
import jax.numpy as jnp
from jax.experimental import pallas as pl
from jax.experimental.pallas import tpu as pltpu


def _linear_kernel(xin_ref, w_ref, bias_ref, xt_ref, *, B, F):
    wt = w_ref[...].T
    for bb in range(B):
        sl = slice(bb * F, (bb + 1) * F)
        y = jnp.dot(xin_ref[:, sl], wt, preferred_element_type=jnp.float32)
        xt_ref[:, sl] = (y + bias_ref[...]).astype(jnp.bfloat16)


def _matmul_kernel(xt_ref, smp_ref, p_ref, out_ref, *, n, blkN):
    i = pl.program_id(0)
    p = p_ref[...].astype(jnp.bfloat16)
    rows = i * blkN + jax.lax.broadcasted_iota(jnp.int32, (blkN, n), 0)
    s_onehot = (smp_ref[...] == rows)
    rowmask = jnp.any(s_onehot, axis=1, keepdims=True)
    p_eff = jnp.where(rowmask, s_onehot.astype(jnp.bfloat16), p)
    out_ref[...] = jnp.dot(p_eff, xt_ref[...],
                           preferred_element_type=jnp.float32)


def kernel(input, P, sample, W, b):
    Bz, n, F = input.shape
    N = P.shape[0]
    BF = Bz * F
    blkN = 512

    xin = input.transpose(1, 0, 2).reshape(n, BF)

    xt = pl.pallas_call(
        functools.partial(_linear_kernel, B=Bz, F=F),
        in_specs=[
            pl.BlockSpec((n, BF), lambda: (0, 0)),
            pl.BlockSpec((F, F), lambda: (0, 0)),
            pl.BlockSpec((1, F), lambda: (0, 0)),
        ],
        out_specs=pl.BlockSpec((n, BF), lambda: (0, 0)),
        out_shape=jax.ShapeDtypeStruct((n, BF), jnp.bfloat16),
    )(xin, W, b.reshape(1, F))

    y2 = pl.pallas_call(
        functools.partial(_matmul_kernel, n=n, blkN=blkN),
        grid=(N // blkN,),
        in_specs=[
            pl.BlockSpec((n, BF), lambda i: (0, 0)),
            pl.BlockSpec((1, n), lambda i: (0, 0)),
            pl.BlockSpec((blkN, n), lambda i: (i, 0)),
        ],
        out_specs=pl.BlockSpec((blkN, BF), lambda i: (i, 0)),
        out_shape=jax.ShapeDtypeStruct((N, BF), jnp.float32),
    )(xt, sample.reshape(1, n).astype(jnp.int32), P)

    return y2.reshape(N, Bz, F).transpose(1, 0, 2)
